# Initial kernel scaffold; baseline (speedup 1.0000x reference)
import jax
import jax.numpy as jnp
from jax.experimental import pallas as pl


def kernel(input_ids, embedding_weights):
    raise NotImplementedError("write your pallas kernel here")


# speedup vs baseline: 1.6848x; 1.6848x over previous
import functools

import jax
import jax.numpy as jnp
from jax import lax
from jax.experimental import pallas as pl
from jax.experimental.pallas import tpu as pltpu
from jax.experimental.pallas import tpu_sc as plsc

D = 64
NW = 32
IDX_W = 128
CHUNK = 1024
N_SUB = CHUNK // IDX_W


@functools.lru_cache(maxsize=None)
def _make_kernel(total: int):
    per_w = total // NW
    steps = per_w // CHUNK
    mesh = plsc.VectorSubcoreMesh(core_axis_name="c", subcore_axis_name="s")

    @functools.partial(
        pl.kernel,
        mesh=mesh,
        out_type=jax.ShapeDtypeStruct((total, D), jnp.float32),
        compiler_params=pltpu.CompilerParams(use_tc_tiling_on_sc=False),
        scratch_types=[
            pltpu.VMEM((N_SUB, IDX_W), jnp.int32),
            pltpu.VMEM((CHUNK, D), jnp.float32),
            pltpu.SemaphoreType.DMA,
        ],
    )
    def emb(idx_hbm, table_hbm, out_hbm, idx_v, rows_v, sem):
        wid = lax.axis_index("s") * 2 + lax.axis_index("c")
        base = wid * per_w

        def step(g, carry):
            off = pl.multiple_of(base + g * CHUNK, CHUNK)
            row = pl.multiple_of(off // IDX_W, 8)
            pltpu.sync_copy(idx_hbm.at[pl.ds(row, N_SUB)], idx_v)
            for j in range(N_SUB):
                pltpu.async_copy(
                    table_hbm.at[idx_v.at[j]],
                    rows_v.at[pl.ds(j * IDX_W, IDX_W)],
                    sem,
                ).wait()
            pltpu.sync_copy(rows_v, out_hbm.at[pl.ds(off, CHUNK)])
            return carry

        lax.fori_loop(0, steps, step, 0)

    return emb


def kernel(input_ids, embedding_weights):
    b, h = input_ids.shape
    total = b * h
    idx2d = input_ids.astype(jnp.int32).reshape(total // IDX_W, IDX_W)
    out = _make_kernel(total)(idx2d, embedding_weights)
    return out.reshape(b, h, D)
